# baseline (speedup 1.0000x reference)
import functools

import jax
import jax.numpy as jnp
from jax.experimental import pallas as pl
from jax.experimental.pallas import tpu as pltpu

PACK = 2


def _silu(v):
    t = 0.5 * v
    return t + t * jnp.tanh(t)


def _mbconv_kernel(x_ref, we_ref, s1_ref, b1_ref, wd_ref, s2_ref, b2_ref,
                   wse1_ref, bse1_ref, wse2_ref, bse2_ref, wp_ref, s3_ref,
                   b3_ref, o_ref, halo_ref, *, K, H, W, LEFT):
    pad = (K - 1) // 2
    C = we_ref.shape[1]
    Cin = we_ref.shape[0]
    Cout = wp_ref.shape[1]
    HW = H * W

    @pl.when(pl.program_id(0) == 0)
    def _zero_halo():
        halo_ref[...] = jnp.zeros_like(halo_ref)

    x = x_ref[0]

    we = we_ref[...] * s1_ref[...]
    b1 = b1_ref[...]
    for p in range(PACK):
        y = jax.lax.dot_general(x[p * Cin:(p + 1) * Cin], we,
                                (((0,), (0,)), ((), ())),
                                preferred_element_type=jnp.float32)
        y = _silu(y + b1)
        halo_ref[pad:pad + H, LEFT:LEFT + W, p * C:(p + 1) * C] = (
            y.reshape(H, W, C))

    acc = None
    for kh in range(K):
        for kw in range(K):
            col = LEFT - pad + kw
            w2 = wd_ref[kh, kw, :] * s2_ref[0]
            wtap = jnp.concatenate([w2] * PACK)
            t = halo_ref[kh:kh + H, col:col + W, :] * wtap
            acc = t if acc is None else acc + t
    b2 = jnp.concatenate([b2_ref[0]] * PACK)
    z = _silu(acc + b2)
    C2 = PACK * C

    pooled = jnp.mean(z.reshape(HW, C2), axis=0, keepdims=True)
    ses = []
    for p in range(PACK):
        h = jnp.dot(pooled[:, p * C:(p + 1) * C], wse1_ref[...],
                    preferred_element_type=jnp.float32) + bse1_ref[...]
        h = _silu(h)
        g = jnp.dot(h, wse2_ref[...],
                    preferred_element_type=jnp.float32) + bse2_ref[...]
        ses.append(0.5 + 0.5 * jnp.tanh(0.5 * g))
    zz = z.reshape(HW, C2) * jnp.concatenate(ses, axis=1)

    wp = wp_ref[...] * s3_ref[...]
    b3 = b3_ref[...]
    for p in range(PACK):
        ot = jax.lax.dot_general(wp, zz[:, p * C:(p + 1) * C],
                                 (((0,), (1,)), ((), ())),
                                 preferred_element_type=jnp.float32)
        o_ref[0, p * Cout:(p + 1) * Cout] = (
            ot + b3 + x[p * Cin:(p + 1) * Cin]).astype(o_ref.dtype)


def kernel(x, w_exp, s1, b1, w_dw, s2, b2, w_se1, b_se1, w_se2, b_se2,
           w_proj, s3, b3):
    N, Cin, H, W = x.shape
    Cexp = w_exp.shape[1]
    Cout = w_proj.shape[1]
    Csq = w_se1.shape[1]
    K = w_dw.shape[0]
    HW = H * W
    pad = (K - 1) // 2
    LEFT = max(8, 8 * pl.cdiv(pad, 8))
    Hp = H + 2 * pad
    Wp = LEFT + W + pad
    NP = N // PACK

    x_blk = x.reshape(NP, PACK * Cin, HW)
    row = lambda v: v.reshape(1, -1)

    out = pl.pallas_call(
        functools.partial(_mbconv_kernel, K=K, H=H, W=W, LEFT=LEFT),
        out_shape=jax.ShapeDtypeStruct((NP, PACK * Cout, HW), x.dtype),
        grid=(NP,),
        in_specs=[
            pl.BlockSpec((1, PACK * Cin, HW), lambda n: (n, 0, 0)),
            pl.BlockSpec((Cin, Cexp), lambda n: (0, 0)),
            pl.BlockSpec((1, Cexp), lambda n: (0, 0)),
            pl.BlockSpec((1, Cexp), lambda n: (0, 0)),
            pl.BlockSpec((K, K, Cexp), lambda n: (0, 0, 0)),
            pl.BlockSpec((1, Cexp), lambda n: (0, 0)),
            pl.BlockSpec((1, Cexp), lambda n: (0, 0)),
            pl.BlockSpec((Cexp, Csq), lambda n: (0, 0)),
            pl.BlockSpec((1, Csq), lambda n: (0, 0)),
            pl.BlockSpec((Csq, Cexp), lambda n: (0, 0)),
            pl.BlockSpec((1, Cexp), lambda n: (0, 0)),
            pl.BlockSpec((Cexp, Cout), lambda n: (0, 0)),
            pl.BlockSpec((1, Cout), lambda n: (0, 0)),
            pl.BlockSpec((Cout, 1), lambda n: (0, 0)),
        ],
        out_specs=pl.BlockSpec((1, PACK * Cout, HW), lambda n: (n, 0, 0)),
        scratch_shapes=[pltpu.VMEM((Hp, Wp, PACK * Cexp), jnp.float32)],
        compiler_params=pltpu.CompilerParams(
            dimension_semantics=("arbitrary",)),
    )(x_blk, w_exp, row(s1), row(b1), w_dw, row(s2), row(b2),
      w_se1, row(b_se1), w_se2, row(b_se2), w_proj, row(s3),
      b3.reshape(Cout, 1))
    return out.reshape(N, Cout, H, W)
